# Initial kernel scaffold; baseline (speedup 1.0000x reference)
import jax
import jax.numpy as jnp
from jax.experimental import pallas as pl


def kernel(v_x, v_up_index, v_down_index, v_up_attr, v_down_attr, e_x, e_up_index, e_down_index, e_up_attr, e_down_attr, t_x, t_up_index, t_down_index, t_up_attr, t_down_attr):
    raise NotImplementedError("write your pallas kernel here")


# speedup vs baseline: 3.2782x; 3.2782x over previous
import functools

import jax
import jax.numpy as jnp
from jax import lax
from jax.experimental import pallas as pl
from jax.experimental.pallas import tpu as pltpu
from jax.experimental.pallas import tpu_sc as plsc

_N = 10000
_E = 320000
_D = 128
_C = 128
_NSUB = 16
_NCHUNKS = _E // _C
_RPT = (_N // _NSUB) & ~7
_TAIL = _N - _NSUB * _RPT


def _sc_body(v_x, v_us, v_ud, v_ds, v_dd, v_ua, v_da,
             e_x, e_us, e_ud, e_ds, e_dd, e_ua, e_da,
             t_x, t_us, t_ud, t_ds, t_dd, t_ua, t_da,
             zeros_hbm,
             o_v, o_e, o_t,
             acc, idx_s, idx_d, rows, attrb, gsem):
    cid = lax.axis_index("c")
    sid = lax.axis_index("s")
    row0 = sid * _RPT

    def run_dir(x_hbm, src_hbm, dst_hbm, attr_hbm):
        n_i = _NCHUNKS // _NSUB + jnp.where(sid < (_NCHUNKS % _NSUB), 1, 0)

        def body(i, carry):
            off = (sid + i * _NSUB) * _C
            pltpu.sync_copy(src_hbm.at[pl.ds(off, _C)], idx_s)
            pltpu.sync_copy(dst_hbm.at[pl.ds(off, _C)], idx_d)
            pltpu.async_copy(x_hbm.at[idx_s], rows, gsem).wait()
            pltpu.sync_copy(attr_hbm.at[pl.ds(off, _C)], attrb)
            pltpu.sync_copy(rows, acc.at[idx_d], add=True)
            pltpu.sync_copy(attrb, acc.at[idx_d], add=True)
            return carry

        lax.fori_loop(0, n_i, body, 0)

    def level(x_hbm, us, ud, ds_, dd, ua, da, out_hbm):
        pltpu.sync_copy(zeros_hbm.at[pl.ds(row0, _RPT)],
                        acc.at[pl.ds(row0, _RPT)])

        @pl.when(sid == _NSUB - 1)
        def _():
            pltpu.sync_copy(zeros_hbm.at[pl.ds(_NSUB * _RPT, _TAIL)],
                            acc.at[pl.ds(_NSUB * _RPT, _TAIL)])

        plsc.subcore_barrier()

        @pl.when(cid == 0)
        def _():
            run_dir(x_hbm, us, ud, ua)

        @pl.when(cid == 1)
        def _():
            run_dir(x_hbm, ds_, dd, da)

        plsc.subcore_barrier()
        pltpu.sync_copy(acc.at[pl.ds(row0, _RPT)],
                        out_hbm.at[cid, pl.ds(row0, _RPT)])

        @pl.when(sid == _NSUB - 1)
        def _():
            pltpu.sync_copy(acc.at[pl.ds(_NSUB * _RPT, _TAIL)],
                            out_hbm.at[cid, pl.ds(_NSUB * _RPT, _TAIL)])

        plsc.subcore_barrier()

    level(v_x, v_us, v_ud, v_ds, v_dd, v_ua, v_da, o_v)
    level(e_x, e_us, e_ud, e_ds, e_dd, e_ua, e_da, o_e)
    level(t_x, t_us, t_ud, t_ds, t_dd, t_ua, t_da, o_t)


_sc_mp = functools.partial(
    pl.kernel,
    out_type=[jax.ShapeDtypeStruct((2, _N, _D), jnp.float32)] * 3,
    mesh=plsc.VectorSubcoreMesh(core_axis_name="c", subcore_axis_name="s"),
    scratch_types=[
        pltpu.VMEM_SHARED((_N, _D), jnp.float32),
        pltpu.VMEM((_C,), jnp.int32),
        pltpu.VMEM((_C,), jnp.int32),
        pltpu.VMEM((_C, _D), jnp.float32),
        pltpu.VMEM((_C, _D), jnp.float32),
        pltpu.SemaphoreType.DMA,
    ],
)(_sc_body)


def _combine_body(pv, pe, pt, ov, oe, ot):
    ov[...] = pv[0] + pv[1]
    oe[...] = pe[0] + pe[1]
    ot[...] = pt[0] + pt[1]


def _combine(pv, pe, pt):
    b = 1000
    return pl.pallas_call(
        _combine_body,
        grid=(_N // b,),
        in_specs=[pl.BlockSpec((2, b, _D), lambda i: (0, i, 0))] * 3,
        out_specs=[pl.BlockSpec((b, _D), lambda i: (i, 0))] * 3,
        out_shape=[jax.ShapeDtypeStruct((_N, _D), jnp.float32)] * 3,
    )(pv, pe, pt)


def kernel(v_x, v_up_index, v_down_index, v_up_attr, v_down_attr,
           e_x, e_up_index, e_down_index, e_up_attr, e_down_attr,
           t_x, t_up_index, t_down_index, t_up_attr, t_down_attr):
    zeros = jnp.zeros((_N, _D), jnp.float32)
    pv, pe, pt = _sc_mp(
        v_x, v_up_index[0], v_up_index[1], v_down_index[0], v_down_index[1],
        v_up_attr, v_down_attr,
        e_x, e_up_index[0], e_up_index[1], e_down_index[0], e_down_index[1],
        e_up_attr, e_down_attr,
        t_x, t_up_index[0], t_up_index[1], t_down_index[0], t_down_index[1],
        t_up_attr, t_down_attr,
        zeros)
    return _combine(pv, pe, pt)


# speedup vs baseline: 4.6119x; 1.4068x over previous
import functools

import jax
import jax.numpy as jnp
from jax import lax
from jax.experimental import pallas as pl
from jax.experimental.pallas import tpu as pltpu
from jax.experimental.pallas import tpu_sc as plsc

_N = 10000
_E = 320000
_D = 128
_C = 128
_NSUB = 16
_NB = 2
_NCHUNKS = _E // _C
_NPT = (_NCHUNKS // _NSUB) // _NB * _NB
_NTAIL = _NCHUNKS - _NSUB * _NPT
_RPT = (_N // _NSUB) & ~7
_TAIL = _N - _NSUB * _RPT


def _sc_body(v_x, v_ui, v_di, v_ua, v_da,
             e_x, e_ui, e_di, e_ua, e_da,
             t_x, t_ui, t_di, t_ua, t_da,
             zeros_hbm,
             o_v, o_e, o_t,
             acc, idxb, rows,
             *sems):
    sem_i = sems[0:_NB]
    sem_g = sems[_NB:2 * _NB]
    sem_s = sems[2 * _NB:3 * _NB]
    sem_a = sems[3 * _NB:4 * _NB]
    cid = lax.axis_index("c")
    sid = lax.axis_index("s")
    row0 = sid * _RPT

    def chunk_serial(x_hbm, idx_hbm, attr_hbm, off):
        pltpu.sync_copy(idx_hbm.at[:, pl.ds(off, _C)], idxb.at[0])
        pltpu.async_copy(x_hbm.at[idxb.at[0, 0]], rows.at[0], sem_g[0]).wait()
        pltpu.sync_copy(rows.at[0], acc.at[idxb.at[0, 1]], add=True)
        pltpu.sync_copy(attr_hbm.at[pl.ds(off, _C)], rows.at[0])
        pltpu.sync_copy(rows.at[0], acc.at[idxb.at[0, 1]], add=True)

    def run_dir(x_hbm, idx_hbm, attr_hbm):
        def body(p, carry):
            i0 = p * _NB
            offs = [(sid + (i0 + b) * _NSUB) * _C for b in range(_NB)]
            ics = [pltpu.async_copy(idx_hbm.at[:, pl.ds(offs[b], _C)],
                                    idxb.at[b], sem_i[b])
                   for b in range(_NB)]
            gs = []
            for b in range(_NB):
                ics[b].wait()
                gs.append(pltpu.async_copy(x_hbm.at[idxb.at[b, 0]],
                                           rows.at[b], sem_g[b]))
            s1s = []
            for b in range(_NB):
                gs[b].wait()
                s1s.append(pltpu.async_copy(rows.at[b], acc.at[idxb.at[b, 1]],
                                            sem_s[b], add=True))
            acs = []
            for b in range(_NB):
                s1s[b].wait()
                acs.append(pltpu.async_copy(attr_hbm.at[pl.ds(offs[b], _C)],
                                            rows.at[b], sem_a[b]))
            s2s = []
            for b in range(_NB):
                acs[b].wait()
                s2s.append(pltpu.async_copy(rows.at[b], acc.at[idxb.at[b, 1]],
                                            sem_s[b], add=True))
            for b in range(_NB):
                s2s[b].wait()
            return carry

        lax.fori_loop(0, _NPT // _NB, body, 0)

        @pl.when(sid < _NTAIL)
        def _():
            chunk_serial(x_hbm, idx_hbm, attr_hbm, (sid + _NPT * _NSUB) * _C)

    def level(x_hbm, ui, di, ua, da, out_hbm):
        pltpu.sync_copy(zeros_hbm.at[pl.ds(row0, _RPT)],
                        acc.at[pl.ds(row0, _RPT)])

        @pl.when(sid == _NSUB - 1)
        def _():
            pltpu.sync_copy(zeros_hbm.at[pl.ds(_NSUB * _RPT, _TAIL)],
                            acc.at[pl.ds(_NSUB * _RPT, _TAIL)])

        plsc.subcore_barrier()

        @pl.when(cid == 0)
        def _():
            run_dir(x_hbm, ui, ua)

        @pl.when(cid == 1)
        def _():
            run_dir(x_hbm, di, da)

        plsc.subcore_barrier()
        pltpu.sync_copy(acc.at[pl.ds(row0, _RPT)],
                        out_hbm.at[cid, pl.ds(row0, _RPT)])

        @pl.when(sid == _NSUB - 1)
        def _():
            pltpu.sync_copy(acc.at[pl.ds(_NSUB * _RPT, _TAIL)],
                            out_hbm.at[cid, pl.ds(_NSUB * _RPT, _TAIL)])

        plsc.subcore_barrier()

    level(v_x, v_ui, v_di, v_ua, v_da, o_v)
    level(e_x, e_ui, e_di, e_ua, e_da, o_e)
    level(t_x, t_ui, t_di, t_ua, t_da, o_t)


_sc_mp = functools.partial(
    pl.kernel,
    out_type=[jax.ShapeDtypeStruct((2, _N, _D), jnp.float32)] * 3,
    mesh=plsc.VectorSubcoreMesh(core_axis_name="c", subcore_axis_name="s"),
    scratch_types=[
        pltpu.VMEM_SHARED((_N, _D), jnp.float32),
        pltpu.VMEM((_NB, 2, _C), jnp.int32),
        pltpu.VMEM((_NB, _C, _D), jnp.float32),
    ] + [pltpu.SemaphoreType.DMA] * (4 * _NB),
)(_sc_body)


def _combine_body(pv, pe, pt, ov, oe, ot):
    ov[...] = pv[0] + pv[1]
    oe[...] = pe[0] + pe[1]
    ot[...] = pt[0] + pt[1]


def _combine(pv, pe, pt):
    b = 1000
    return pl.pallas_call(
        _combine_body,
        grid=(_N // b,),
        in_specs=[pl.BlockSpec((2, b, _D), lambda i: (0, i, 0))] * 3,
        out_specs=[pl.BlockSpec((b, _D), lambda i: (i, 0))] * 3,
        out_shape=[jax.ShapeDtypeStruct((_N, _D), jnp.float32)] * 3,
    )(pv, pe, pt)


def kernel(v_x, v_up_index, v_down_index, v_up_attr, v_down_attr,
           e_x, e_up_index, e_down_index, e_up_attr, e_down_attr,
           t_x, t_up_index, t_down_index, t_up_attr, t_down_attr):
    zeros = jnp.zeros((_N, _D), jnp.float32)
    pv, pe, pt = _sc_mp(
        v_x, v_up_index, v_down_index, v_up_attr, v_down_attr,
        e_x, e_up_index, e_down_index, e_up_attr, e_down_attr,
        t_x, t_up_index, t_down_index, t_up_attr, t_down_attr,
        zeros)
    return _combine(pv, pe, pt)


# speedup vs baseline: 5.2634x; 1.1413x over previous
import functools

import jax
import jax.numpy as jnp
from jax import lax
from jax.experimental import pallas as pl
from jax.experimental.pallas import tpu as pltpu
from jax.experimental.pallas import tpu_sc as plsc

_N = 10000
_E = 320000
_D = 128
_C = 128
_NSUB = 16
_NB = 3
_NCHUNKS = _E // _C
_NPT = (_NCHUNKS // _NSUB) // _NB * _NB
_NTAIL = _NCHUNKS - _NSUB * _NPT
_RPT = (_N // _NSUB) & ~7
_TAIL = _N - _NSUB * _RPT


def _sc_body(v_x, v_ui, v_di, v_ua, v_da,
             e_x, e_ui, e_di, e_ua, e_da,
             t_x, t_ui, t_di, t_ua, t_da,
             zeros_hbm,
             o_v, o_e, o_t,
             acc, idxb, rows,
             *sems):
    sem_i = sems[0:_NB]
    sem_g = sems[_NB:2 * _NB]
    sem_s = sems[2 * _NB:3 * _NB]
    sem_a = sems[3 * _NB:4 * _NB]
    cid = lax.axis_index("c")
    sid = lax.axis_index("s")
    row0 = sid * _RPT

    def chunk_serial(x_hbm, idx_hbm, attr_hbm, off):
        pltpu.sync_copy(idx_hbm.at[:, pl.ds(off, _C)], idxb.at[0])
        pltpu.async_copy(x_hbm.at[idxb.at[0, 0]], rows.at[0], sem_g[0]).wait()
        pltpu.sync_copy(rows.at[0], acc.at[idxb.at[0, 1]], add=True)
        pltpu.sync_copy(attr_hbm.at[pl.ds(off, _C)], rows.at[0])
        pltpu.sync_copy(rows.at[0], acc.at[idxb.at[0, 1]], add=True)

    def run_dir(x_hbm, idx_hbm, attr_hbm):
        def body(p, carry):
            i0 = p * _NB
            offs = [(sid + (i0 + b) * _NSUB) * _C for b in range(_NB)]
            ics = [pltpu.async_copy(idx_hbm.at[:, pl.ds(offs[b], _C)],
                                    idxb.at[b], sem_i[b])
                   for b in range(_NB)]
            gs = []
            for b in range(_NB):
                ics[b].wait()
                gs.append(pltpu.async_copy(x_hbm.at[idxb.at[b, 0]],
                                           rows.at[b], sem_g[b]))
            s1s = []
            for b in range(_NB):
                gs[b].wait()
                s1s.append(pltpu.async_copy(rows.at[b], acc.at[idxb.at[b, 1]],
                                            sem_s[b], add=True))
            acs = []
            for b in range(_NB):
                s1s[b].wait()
                acs.append(pltpu.async_copy(attr_hbm.at[pl.ds(offs[b], _C)],
                                            rows.at[b], sem_a[b]))
            s2s = []
            for b in range(_NB):
                acs[b].wait()
                s2s.append(pltpu.async_copy(rows.at[b], acc.at[idxb.at[b, 1]],
                                            sem_s[b], add=True))
            for b in range(_NB):
                s2s[b].wait()
            return carry

        lax.fori_loop(0, _NPT // _NB, body, 0)

        @pl.when(sid < _NTAIL)
        def _():
            chunk_serial(x_hbm, idx_hbm, attr_hbm, (sid + _NPT * _NSUB) * _C)

    def level(x_hbm, ui, di, ua, da, out_hbm):
        pltpu.sync_copy(zeros_hbm.at[pl.ds(row0, _RPT)],
                        acc.at[pl.ds(row0, _RPT)])

        @pl.when(sid == _NSUB - 1)
        def _():
            pltpu.sync_copy(zeros_hbm.at[pl.ds(_NSUB * _RPT, _TAIL)],
                            acc.at[pl.ds(_NSUB * _RPT, _TAIL)])

        plsc.subcore_barrier()

        @pl.when(cid == 0)
        def _():
            run_dir(x_hbm, ui, ua)

        @pl.when(cid == 1)
        def _():
            run_dir(x_hbm, di, da)

        plsc.subcore_barrier()
        pltpu.sync_copy(acc.at[pl.ds(row0, _RPT)],
                        out_hbm.at[cid, pl.ds(row0, _RPT)])

        @pl.when(sid == _NSUB - 1)
        def _():
            pltpu.sync_copy(acc.at[pl.ds(_NSUB * _RPT, _TAIL)],
                            out_hbm.at[cid, pl.ds(_NSUB * _RPT, _TAIL)])

        plsc.subcore_barrier()

    level(v_x, v_ui, v_di, v_ua, v_da, o_v)
    level(e_x, e_ui, e_di, e_ua, e_da, o_e)
    level(t_x, t_ui, t_di, t_ua, t_da, o_t)


_sc_mp = functools.partial(
    pl.kernel,
    out_type=[jax.ShapeDtypeStruct((2, _N, _D), jnp.float32)] * 3,
    mesh=plsc.VectorSubcoreMesh(core_axis_name="c", subcore_axis_name="s"),
    scratch_types=[
        pltpu.VMEM_SHARED((_N, _D), jnp.float32),
        pltpu.VMEM((_NB, 2, _C), jnp.int32),
        pltpu.VMEM((_NB, _C, _D), jnp.float32),
    ] + [pltpu.SemaphoreType.DMA] * (4 * _NB),
)(_sc_body)


def _combine_body(pv, pe, pt, ov, oe, ot):
    ov[...] = pv[0] + pv[1]
    oe[...] = pe[0] + pe[1]
    ot[...] = pt[0] + pt[1]


def _combine(pv, pe, pt):
    b = 1000
    return pl.pallas_call(
        _combine_body,
        grid=(_N // b,),
        in_specs=[pl.BlockSpec((2, b, _D), lambda i: (0, i, 0))] * 3,
        out_specs=[pl.BlockSpec((b, _D), lambda i: (i, 0))] * 3,
        out_shape=[jax.ShapeDtypeStruct((_N, _D), jnp.float32)] * 3,
    )(pv, pe, pt)


def kernel(v_x, v_up_index, v_down_index, v_up_attr, v_down_attr,
           e_x, e_up_index, e_down_index, e_up_attr, e_down_attr,
           t_x, t_up_index, t_down_index, t_up_attr, t_down_attr):
    zeros = jnp.zeros((_N, _D), jnp.float32)
    pv, pe, pt = _sc_mp(
        v_x, v_up_index, v_down_index, v_up_attr, v_down_attr,
        e_x, e_up_index, e_down_index, e_up_attr, e_down_attr,
        t_x, t_up_index, t_down_index, t_up_attr, t_down_attr,
        zeros)
    return _combine(pv, pe, pt)
